# Initial kernel scaffold; baseline (speedup 1.0000x reference)
import jax
import jax.numpy as jnp
from jax import lax
from jax.experimental import pallas as pl
from jax.experimental.pallas import tpu as pltpu
from jax.experimental.pallas import tpu_sc as plsc

B = 32
N_POINTS = 100000
N_OUT = 50000
FP_LEN = 4
NNZ = 1600000
NUM_ROWS = N_OUT * FP_LEN

RPB = 400
NB = NUM_ROWS // RPB
NTILES = 32
KMAX = -(-NB // NTILES)
W = 128
NWIN = W // 16
PADE = 2 * W
BND_PAD = 512


def _sc_body(ut_hbm, qn_hbm, qr_hbm, qv_hbm, bnd_hbm, out_hbm,
             bnd_v, idx_v, qr_v, qv_v, rows_v, acc_v):
    iota = lax.iota(jnp.int32, 16)
    wid = lax.axis_index("s") * 2 + lax.axis_index("c")

    pltpu.sync_copy(bnd_hbm, bnd_v)

    @pl.loop(0, KMAX)
    def _round(k):
        blk = wid + NTILES * k

        @pl.when(blk < NB)
        def _process():
            r0 = blk * RPB
            e0 = jnp.max(plsc.load_gather(bnd_v, [jnp.full((16,), blk, jnp.int32)]))
            e1 = jnp.max(plsc.load_gather(bnd_v, [jnp.full((16,), blk + 1, jnp.int32)]))

            zeros16 = jnp.zeros((16,), jnp.float32)

            @pl.loop(0, B)
            def _zb(bi):
                @pl.loop(0, RPB // 16)
                def _zj(j):
                    acc_v[bi, pl.ds(j * 16, 16)] = zeros16

            e_base = pl.multiple_of(e0 - lax.rem(e0, 16), 16)
            nch = lax.div(e1 - e_base + (W - 1), W)
            e0f = jnp.full((16,), e0, jnp.int32)
            e1f = jnp.full((16,), e1, jnp.int32)
            r0f = jnp.full((16,), r0, jnp.int32)

            def _chunk(c, carry):
                ec = pl.multiple_of(e_base + c * W, 16)
                pltpu.sync_copy(qn_hbm.at[pl.ds(ec, W)], idx_v)
                pltpu.sync_copy(qr_hbm.at[pl.ds(ec, W)], qr_v)
                pltpu.sync_copy(qv_hbm.at[pl.ds(ec, W)], qv_v)
                pltpu.sync_copy(ut_hbm.at[idx_v], rows_v)

                @pl.loop(0, NWIN)
                def _window(wd):
                    base = wd * 16
                    wvec = qv_v[pl.ds(base, 16)]
                    rvec = qr_v[pl.ds(base, 16)]
                    rl = rvec - r0f
                    gid = jnp.full((16,), ec + base, jnp.int32) + iota
                    mask = (gid >= e0f) & (gid < e1f)
                    eoff = jnp.full((16,), base, jnp.int32) + iota
                    for t in range(B):
                        btl = (iota + jnp.int32(t)) & jnp.int32(B - 1)
                        col = plsc.load_gather(rows_v, [eoff, btl])
                        plsc.addupdate_scatter(acc_v, [btl, rl], col * wvec,
                                               mask=mask)

                return carry

            lax.fori_loop(0, nch, _chunk, 0)

            pltpu.sync_copy(acc_v, out_hbm.at[:, pl.ds(r0, RPB)])


@jax.jit
def kernel(U, Qrow, Qneigh, Qval):
    UT = jnp.transpose(U)

    row_starts = jnp.arange(NB + 1, dtype=jnp.int32) * RPB
    bounds = jnp.searchsorted(Qrow, row_starts, side="left").astype(jnp.int32)
    bounds = jnp.pad(bounds, (0, BND_PAD - (NB + 1)), constant_values=NNZ)

    qn = jnp.pad(Qneigh, (0, PADE), constant_values=0)
    qr = jnp.pad(Qrow, (0, PADE), constant_values=NUM_ROWS)
    qv = jnp.pad(Qval, (0, PADE), constant_values=0.0)

    mesh = plsc.VectorSubcoreMesh(core_axis_name="c", subcore_axis_name="s")
    out = pl.kernel(
        _sc_body,
        out_type=jax.ShapeDtypeStruct((B, NUM_ROWS), jnp.float32),
        mesh=mesh,
        scratch_types=[
            pltpu.VMEM((BND_PAD,), jnp.int32),
            pltpu.VMEM((W,), jnp.int32),
            pltpu.VMEM((W,), jnp.int32),
            pltpu.VMEM((W,), jnp.float32),
            pltpu.VMEM((W, B), jnp.float32),
            pltpu.VMEM((B, RPB), jnp.float32),
        ],
    )(UT, qn, qr, qv, bounds)

    return jnp.reshape(out, (B, N_OUT, FP_LEN))


# speedup vs baseline: 3.2055x; 3.2055x over previous
import dataclasses

import jax
import jax.numpy as jnp
from jax import lax
from jax.experimental import pallas as pl
from jax.experimental.pallas import tpu as pltpu
from jax.experimental.pallas import tpu_sc as plsc

B = 32
N_POINTS = 100000
N_OUT = 50000
FP_LEN = 4
NNZ = 1600000
NUM_ROWS = N_OUT * FP_LEN

RPB = 400
NB = NUM_ROWS // RPB
NTILES = 32
KMAX = -(-NB // NTILES)
W = 128
NWIN = W // 16
PADE = 2 * W
BND_PAD = 512


def _sc_body(ut_hbm, qn_hbm, qr_hbm, qv_hbm, bnd_hbm, out_hbm,
             bnd_v, idx_v, qr_v, qv_v, rows_v, acc_v, dma_sem):
    iota = lax.iota(jnp.int32, 16)
    wid = lax.axis_index("s") * 2 + lax.axis_index("c")

    pltpu.sync_copy(bnd_hbm, bnd_v)

    @pl.loop(0, KMAX)
    def _round(k):
        blk = wid + NTILES * k

        @pl.when(blk < NB)
        def _process():
            r0 = blk * RPB
            e0 = jnp.max(plsc.load_gather(bnd_v, [jnp.full((16,), blk, jnp.int32)]))
            e1 = jnp.max(plsc.load_gather(bnd_v, [jnp.full((16,), blk + 1, jnp.int32)]))

            zeros16 = jnp.zeros((16,), jnp.float32)

            @pl.loop(0, B)
            def _zb(bi):
                @pl.loop(0, RPB // 16)
                def _zj(j):
                    acc_v[bi, pl.ds(j * 16, 16)] = zeros16

            e_base = pl.multiple_of(e0 - lax.rem(e0, 16), 16)
            nch = lax.div(e1 - e_base + (W - 1), W)
            e0f = jnp.full((16,), e0, jnp.int32)
            e1f = jnp.full((16,), e1, jnp.int32)
            r0f = jnp.full((16,), r0, jnp.int32)

            def _chunk(c, carry):
                ec = pl.multiple_of(e_base + c * W, 16)
                pltpu.sync_copy(qn_hbm.at[pl.ds(ec, W)], idx_v)
                pltpu.sync_copy(qr_hbm.at[pl.ds(ec, W)], qr_v)
                pltpu.sync_copy(qv_hbm.at[pl.ds(ec, W)], qv_v)
                pltpu.sync_copy(ut_hbm.at[idx_v], rows_v)

                @pl.loop(0, NWIN)
                def _window(wd):
                    base = wd * 16
                    wvec = qv_v[pl.ds(base, 16)]
                    rvec = qr_v[pl.ds(base, 16)]
                    rl = rvec - r0f
                    gid = jnp.full((16,), ec + base, jnp.int32) + iota
                    mask = (gid >= e0f) & (gid < e1f)
                    eoff = jnp.full((16,), base, jnp.int32) + iota
                    for t in range(B):
                        btl = (iota + jnp.int32(t)) & jnp.int32(B - 1)
                        col = plsc.load_gather(rows_v, [eoff, btl])
                        plsc.addupdate_scatter(acc_v, [btl, rl], col * wvec,
                                               mask=mask)

                return carry

            lax.fori_loop(0, nch, _chunk, 0)

            copies = [
                pltpu.async_copy(acc_v.at[b],
                                 out_hbm.at[pl.ds(b * NUM_ROWS + r0, RPB)],
                                 dma_sem)
                for b in range(B)
            ]
            for c in copies:
                c.wait()


@jax.jit
def kernel(U, Qrow, Qneigh, Qval):
    UT = jnp.transpose(U)

    row_starts = jnp.arange(NB + 1, dtype=jnp.int32) * RPB
    bounds = jnp.searchsorted(Qrow, row_starts, side="left").astype(jnp.int32)
    bounds = jnp.pad(bounds, (0, BND_PAD - (NB + 1)), constant_values=NNZ)

    qn = jnp.pad(Qneigh, (0, PADE), constant_values=0)
    qr = jnp.pad(Qrow, (0, PADE), constant_values=NUM_ROWS)
    qv = jnp.pad(Qval, (0, PADE), constant_values=0.0)

    mesh = plsc.VectorSubcoreMesh(core_axis_name="c", subcore_axis_name="s")
    cp = pltpu.CompilerParams()
    if "needs_layout_passes" in pltpu.CompilerParams.__dataclass_fields__:
        cp = dataclasses.replace(cp, needs_layout_passes=False)
    cp = dataclasses.replace(cp, use_tc_tiling_on_sc=False)
    out = pl.kernel(
        _sc_body,
        compiler_params=cp,
        out_type=jax.ShapeDtypeStruct((B * NUM_ROWS,), jnp.float32),
        mesh=mesh,
        scratch_types=[
            pltpu.VMEM((BND_PAD,), jnp.int32),
            pltpu.VMEM((W,), jnp.int32),
            pltpu.VMEM((W,), jnp.int32),
            pltpu.VMEM((W,), jnp.float32),
            pltpu.VMEM((W, B), jnp.float32),
            pltpu.VMEM((B, RPB), jnp.float32),
            pltpu.SemaphoreType.DMA,
        ],
    )(UT, qn, qr, qv, bounds)

    return jnp.reshape(out, (B, N_OUT, FP_LEN))


# speedup vs baseline: 3.3327x; 1.0397x over previous
import dataclasses

import jax
import jax.numpy as jnp
from jax import lax
from jax.experimental import pallas as pl
from jax.experimental.pallas import tpu as pltpu
from jax.experimental.pallas import tpu_sc as plsc

B = 32
N_POINTS = 100000
N_OUT = 50000
FP_LEN = 4
NNZ = 1600000
NUM_ROWS = N_OUT * FP_LEN

RPB = 400
ACC_PITCH = 401
NB = NUM_ROWS // RPB
NTILES = 32
KMAX = -(-NB // NTILES)
W = 128
NWIN = W // 16
PADE = 2 * W
BND_PAD = 512


def _sc_body(ut_hbm, qn_hbm, qr_hbm, qv_hbm, bnd_hbm, out_hbm,
             bnd_v, idx_v, qr_v, qv_v, rows_v, acc_v, dma_sem):
    iota = lax.iota(jnp.int32, 16)
    wid = lax.axis_index("s") * 2 + lax.axis_index("c")

    pltpu.sync_copy(bnd_hbm, bnd_v)

    @pl.loop(0, KMAX)
    def _round(k):
        blk = wid + NTILES * k

        @pl.when(blk < NB)
        def _process():
            r0 = blk * RPB
            e0 = jnp.max(plsc.load_gather(bnd_v, [jnp.full((16,), blk, jnp.int32)]))
            e1 = jnp.max(plsc.load_gather(bnd_v, [jnp.full((16,), blk + 1, jnp.int32)]))

            zeros16 = jnp.zeros((16,), jnp.float32)

            @pl.loop(0, B)
            def _zb(bi):
                @pl.loop(0, RPB // 16)
                def _zj(j):
                    acc_v[bi, pl.ds(j * 16, 16)] = zeros16

            e_base = pl.multiple_of(e0 - lax.rem(e0, 16), 16)
            nch = lax.div(e1 - e_base + (W - 1), W)
            e0f = jnp.full((16,), e0, jnp.int32)
            e1f = jnp.full((16,), e1, jnp.int32)
            r0f = jnp.full((16,), r0, jnp.int32)

            def _chunk(c, carry):
                ec = pl.multiple_of(e_base + c * W, 16)
                pltpu.sync_copy(qn_hbm.at[pl.ds(ec, W)], idx_v)
                pltpu.sync_copy(qr_hbm.at[pl.ds(ec, W)], qr_v)
                pltpu.sync_copy(qv_hbm.at[pl.ds(ec, W)], qv_v)
                pltpu.sync_copy(ut_hbm.at[idx_v], rows_v)

                @pl.loop(0, NWIN)
                def _window(wd):
                    base = wd * 16
                    wvec = qv_v[pl.ds(base, 16)]
                    rvec = qr_v[pl.ds(base, 16)]
                    rl = rvec - r0f
                    gid = jnp.full((16,), ec + base, jnp.int32) + iota
                    mask = (gid >= e0f) & (gid < e1f)
                    eoff = jnp.full((16,), base, jnp.int32) + iota
                    for t in range(B):
                        btl = (iota + jnp.int32(t)) & jnp.int32(B - 1)
                        col = plsc.load_gather(rows_v, [eoff, btl])
                        plsc.addupdate_scatter(acc_v, [btl, rl], col * wvec,
                                               mask=mask)

                return carry

            lax.fori_loop(0, nch, _chunk, 0)

            copies = [
                pltpu.async_copy(acc_v.at[b, pl.ds(0, RPB)],
                                 out_hbm.at[pl.ds(b * NUM_ROWS + r0, RPB)],
                                 dma_sem)
                for b in range(B)
            ]
            for c in copies:
                c.wait()


@jax.jit
def kernel(U, Qrow, Qneigh, Qval):
    UT = jnp.transpose(U)

    row_starts = jnp.arange(NB + 1, dtype=jnp.int32) * RPB
    bounds = jnp.searchsorted(Qrow, row_starts, side="left").astype(jnp.int32)
    bounds = jnp.pad(bounds, (0, BND_PAD - (NB + 1)), constant_values=NNZ)

    qn = jnp.pad(Qneigh, (0, PADE), constant_values=0)
    qr = jnp.pad(Qrow, (0, PADE), constant_values=NUM_ROWS)
    qv = jnp.pad(Qval, (0, PADE), constant_values=0.0)

    mesh = plsc.VectorSubcoreMesh(core_axis_name="c", subcore_axis_name="s")
    cp = pltpu.CompilerParams()
    if "needs_layout_passes" in pltpu.CompilerParams.__dataclass_fields__:
        cp = dataclasses.replace(cp, needs_layout_passes=False)
    cp = dataclasses.replace(cp, use_tc_tiling_on_sc=False)
    out = pl.kernel(
        _sc_body,
        compiler_params=cp,
        out_type=jax.ShapeDtypeStruct((B * NUM_ROWS,), jnp.float32),
        mesh=mesh,
        scratch_types=[
            pltpu.VMEM((BND_PAD,), jnp.int32),
            pltpu.VMEM((W,), jnp.int32),
            pltpu.VMEM((W,), jnp.int32),
            pltpu.VMEM((W,), jnp.float32),
            pltpu.VMEM((W, B), jnp.float32),
            pltpu.VMEM((B, ACC_PITCH), jnp.float32),
            pltpu.SemaphoreType.DMA,
        ],
    )(UT, qn, qr, qv, bounds)

    return jnp.reshape(out, (B, N_OUT, FP_LEN))


# speedup vs baseline: 5.1331x; 1.5402x over previous
import dataclasses

import jax
import jax.numpy as jnp
from jax import lax
from jax.experimental import pallas as pl
from jax.experimental.pallas import tpu as pltpu
from jax.experimental.pallas import tpu_sc as plsc

B = 32
N_POINTS = 100000
N_OUT = 50000
FP_LEN = 4
NNZ = 1600000
NUM_ROWS = N_OUT * FP_LEN

RPB = 400
ACC_PITCH = 401
NB = NUM_ROWS // RPB
NTILES = 32
KMAX = -(-NB // NTILES)
W = 128
NWIN = W // 16
PADE = 2 * W
BND_PAD = 512


def _sc_body(ut_hbm, qn_hbm, qr_hbm, qv_hbm, bnd_hbm, out_hbm,
             bnd_v, idx_v, qr_v, qv_v, rows_v, acc_v, dma_sem):
    iota = lax.iota(jnp.int32, 16)
    wid = lax.axis_index("s") * 2 + lax.axis_index("c")

    pltpu.sync_copy(bnd_hbm, bnd_v)

    @pl.loop(0, KMAX)
    def _round(k):
        blk = wid + NTILES * k

        @pl.when(blk < NB)
        def _process():
            r0 = blk * RPB
            e0 = jnp.max(plsc.load_gather(bnd_v, [jnp.full((16,), blk, jnp.int32)]))
            e1 = jnp.max(plsc.load_gather(bnd_v, [jnp.full((16,), blk + 1, jnp.int32)]))

            zeros16 = jnp.zeros((16,), jnp.float32)

            @pl.loop(0, B)
            def _zb(bi):
                @pl.loop(0, RPB // 16)
                def _zj(j):
                    acc_v[bi, pl.ds(j * 16, 16)] = zeros16

            e_base = pl.multiple_of(e0 - lax.rem(e0, 16), 16)
            nch = lax.div(e1 - e_base + (W - 1), W)
            e0f = jnp.full((16,), e0, jnp.int32)
            e1f = jnp.full((16,), e1, jnp.int32)
            r0f = jnp.full((16,), r0, jnp.int32)

            def _chunk(c, carry):
                ec = pl.multiple_of(e_base + c * W, 16)
                pltpu.sync_copy(qn_hbm.at[pl.ds(ec, W)], idx_v)
                pltpu.sync_copy(qr_hbm.at[pl.ds(ec, W)], qr_v)
                pltpu.sync_copy(qv_hbm.at[pl.ds(ec, W)], qv_v)
                pltpu.sync_copy(ut_hbm.at[idx_v], rows_v)

                @pl.loop(0, NWIN)
                def _window(wd):
                    base = wd * 16
                    wvec = qv_v[pl.ds(base, 16)]
                    rvec = qr_v[pl.ds(base, 16)]
                    rl = rvec - r0f
                    gid = jnp.full((16,), ec + base, jnp.int32) + iota
                    mask = (gid >= e0f) & (gid < e1f)
                    eoff = jnp.full((16,), base, jnp.int32) + iota
                    for t in range(B):
                        btl = (iota + jnp.int32(t)) & jnp.int32(B - 1)
                        col = plsc.load_gather(rows_v, [eoff, btl])
                        plsc.addupdate_scatter(acc_v, [btl, rl], col * wvec,
                                               mask=mask)

                return carry

            lax.fori_loop(0, nch, _chunk, 0)

            copies = [
                pltpu.async_copy(acc_v.at[b, pl.ds(0, RPB)],
                                 out_hbm.at[b, pl.ds(r0, RPB)],
                                 dma_sem)
                for b in range(B)
            ]
            for c in copies:
                c.wait()


@jax.jit
def kernel(U, Qrow, Qneigh, Qval):
    UT = jnp.transpose(U)

    row_starts = jnp.arange(NB + 1, dtype=jnp.int32) * RPB
    bounds = jnp.searchsorted(Qrow, row_starts, side="left").astype(jnp.int32)
    bounds = jnp.pad(bounds, (0, BND_PAD - (NB + 1)), constant_values=NNZ)

    qn = jnp.pad(Qneigh, (0, PADE), constant_values=0)
    qr = jnp.pad(Qrow, (0, PADE), constant_values=NUM_ROWS)
    qv = jnp.pad(Qval, (0, PADE), constant_values=0.0)

    mesh = plsc.VectorSubcoreMesh(core_axis_name="c", subcore_axis_name="s")
    cp = pltpu.CompilerParams()
    if "needs_layout_passes" in pltpu.CompilerParams.__dataclass_fields__:
        cp = dataclasses.replace(cp, needs_layout_passes=False)
    cp = dataclasses.replace(cp, use_tc_tiling_on_sc=False)
    out = pl.kernel(
        _sc_body,
        compiler_params=cp,
        out_type=jax.ShapeDtypeStruct((B, NUM_ROWS), jnp.float32),
        mesh=mesh,
        scratch_types=[
            pltpu.VMEM((BND_PAD,), jnp.int32),
            pltpu.VMEM((W,), jnp.int32),
            pltpu.VMEM((W,), jnp.int32),
            pltpu.VMEM((W,), jnp.float32),
            pltpu.VMEM((W, B), jnp.float32),
            pltpu.VMEM((B, ACC_PITCH), jnp.float32),
            pltpu.SemaphoreType.DMA,
        ],
    )(UT, qn, qr, qv, bounds)

    return jnp.reshape(out, (B, N_OUT, FP_LEN))


# speedup vs baseline: 7.3566x; 1.4332x over previous
import dataclasses

import jax
import jax.numpy as jnp
from jax import lax
from jax.experimental import pallas as pl
from jax.experimental.pallas import tpu as pltpu
from jax.experimental.pallas import tpu_sc as plsc

B = 32
N_POINTS = 100000
N_OUT = 50000
FP_LEN = 4
NNZ = 1600000
NUM_ROWS = N_OUT * FP_LEN

RPB = 400
ACC_PITCH = 401
NB = NUM_ROWS // RPB
NTILES = 32
KMAX = -(-NB // NTILES)
W = 128
S = 1024
NGPS = S // W
NWINS = S // 16
PADE = S
BND_PAD = 512


def _sc_body(ut_hbm, qn_hbm, qr_hbm, qv_hbm, bnd_hbm, out_hbm,
             bnd_v, qn_v, qr_v, qv_v, rows_v, acc_v,
             nsem, lsem, gsem, dma_sem):
    iota = lax.iota(jnp.int32, 16)
    wid = lax.axis_index("s") * 2 + lax.axis_index("c")

    pltpu.sync_copy(bnd_hbm, bnd_v)

    @pl.loop(0, KMAX)
    def _round(k):
        blk = wid + NTILES * k

        @pl.when(blk < NB)
        def _process():
            r0 = blk * RPB
            e0 = jnp.max(plsc.load_gather(bnd_v, [jnp.full((16,), blk, jnp.int32)]))
            e1 = jnp.max(plsc.load_gather(bnd_v, [jnp.full((16,), blk + 1, jnp.int32)]))

            zeros16 = jnp.zeros((16,), jnp.float32)

            @pl.loop(0, B)
            def _zb(bi):
                @pl.loop(0, RPB // 16)
                def _zj(j):
                    acc_v[bi, pl.ds(j * 16, 16)] = zeros16

            e_base = pl.multiple_of(e0 - lax.rem(e0, 16), 16)
            nsc = lax.div(e1 - e_base + (S - 1), S)
            e0f = jnp.full((16,), e0, jnp.int32)
            e1f = jnp.full((16,), e1, jnp.int32)
            r0f = jnp.full((16,), r0, jnp.int32)

            def _super(s, carry):
                ec = pl.multiple_of(e_base + s * S, 16)
                h_qn = pltpu.async_copy(qn_hbm.at[pl.ds(ec, S)], qn_v, nsem)
                h_qr = pltpu.async_copy(qr_hbm.at[pl.ds(ec, S)], qr_v, lsem)
                h_qv = pltpu.async_copy(qv_hbm.at[pl.ds(ec, S)], qv_v, lsem)
                h_qn.wait()
                h_rows = [
                    pltpu.async_copy(
                        ut_hbm.at[qn_v.at[pl.ds(j * W, W)]],
                        rows_v.at[pl.ds(j * W, W), :],
                        gsem)
                    for j in range(NGPS)
                ]
                h_qr.wait()
                h_qv.wait()
                for h in h_rows:
                    h.wait()

                @pl.loop(0, NWINS)
                def _window(wd):
                    base = wd * 16
                    wvec = qv_v[pl.ds(base, 16)]
                    rvec = qr_v[pl.ds(base, 16)]
                    rl = rvec - r0f
                    gid = jnp.full((16,), ec + base, jnp.int32) + iota
                    mask = (gid >= e0f) & (gid < e1f)
                    eoff = jnp.full((16,), base, jnp.int32) + iota
                    for t in range(B):
                        btl = (iota + jnp.int32(t)) & jnp.int32(B - 1)
                        col = plsc.load_gather(rows_v, [eoff, btl])
                        plsc.addupdate_scatter(acc_v, [btl, rl], col * wvec,
                                               mask=mask)

                return carry

            lax.fori_loop(0, nsc, _super, 0)

            copies = [
                pltpu.async_copy(acc_v.at[b, pl.ds(0, RPB)],
                                 out_hbm.at[b, pl.ds(r0, RPB)],
                                 dma_sem)
                for b in range(B)
            ]
            for c in copies:
                c.wait()


@jax.jit
def kernel(U, Qrow, Qneigh, Qval):
    UT = jnp.transpose(U)

    row_starts = jnp.arange(NB + 1, dtype=jnp.int32) * RPB
    bounds = jnp.searchsorted(Qrow, row_starts, side="left").astype(jnp.int32)
    bounds = jnp.pad(bounds, (0, BND_PAD - (NB + 1)), constant_values=NNZ)

    qn = jnp.pad(Qneigh, (0, PADE), constant_values=0)
    qr = jnp.pad(Qrow, (0, PADE), constant_values=NUM_ROWS)
    qv = jnp.pad(Qval, (0, PADE), constant_values=0.0)

    mesh = plsc.VectorSubcoreMesh(core_axis_name="c", subcore_axis_name="s")
    cp = pltpu.CompilerParams()
    if "needs_layout_passes" in pltpu.CompilerParams.__dataclass_fields__:
        cp = dataclasses.replace(cp, needs_layout_passes=False)
    cp = dataclasses.replace(cp, use_tc_tiling_on_sc=False)
    out = pl.kernel(
        _sc_body,
        compiler_params=cp,
        out_type=jax.ShapeDtypeStruct((B, NUM_ROWS), jnp.float32),
        mesh=mesh,
        scratch_types=[
            pltpu.VMEM((BND_PAD,), jnp.int32),
            pltpu.VMEM((S,), jnp.int32),
            pltpu.VMEM((S,), jnp.int32),
            pltpu.VMEM((S,), jnp.float32),
            pltpu.VMEM((S, B), jnp.float32),
            pltpu.VMEM((B, ACC_PITCH), jnp.float32),
            pltpu.SemaphoreType.DMA,
            pltpu.SemaphoreType.DMA,
            pltpu.SemaphoreType.DMA,
            pltpu.SemaphoreType.DMA,
        ],
    )(UT, qn, qr, qv, bounds)

    return jnp.reshape(out, (B, N_OUT, FP_LEN))


# speedup vs baseline: 8.0016x; 1.0877x over previous
import dataclasses

import jax
import jax.numpy as jnp
from jax import lax
from jax.experimental import pallas as pl
from jax.experimental.pallas import tpu as pltpu
from jax.experimental.pallas import tpu_sc as plsc

B = 32
N_POINTS = 100000
N_OUT = 50000
FP_LEN = 4
NNZ = 1600000
NUM_ROWS = N_OUT * FP_LEN

RPB = 400
ACC_PITCH = 401
NB = NUM_ROWS // RPB
NTILES = 32
KMAX = -(-NB // NTILES)
W = 128
S = 1024
NGPS = S // W
NWINS = S // 16
PADE = 2 * S
BND_PAD = 512


def _sc_body(ut_hbm, qn_hbm, qr_hbm, qv_hbm, bnd_hbm, out_hbm,
             bnd_v, qn_a, qr_a, qv_a, rows_a, qn_b, qr_b, qv_b, rows_b,
             acc_v, nsem_a, lsem_a, gsem_a, nsem_b, lsem_b, gsem_b, dma_sem):
    iota = lax.iota(jnp.int32, 16)
    wid = lax.axis_index("s") * 2 + lax.axis_index("c")

    pltpu.async_copy(bnd_hbm, bnd_v, dma_sem).wait()

    bufs = {
        0: (qn_a, qr_a, qv_a, rows_a, nsem_a, lsem_a, gsem_a),
        1: (qn_b, qr_b, qv_b, rows_b, nsem_b, lsem_b, gsem_b),
    }

    def fire_linear(p, ec):
        qn_v, qr_v, qv_v, _, nsem, lsem, _ = bufs[p]
        pltpu.async_copy(qn_hbm.at[pl.ds(ec, S)], qn_v, nsem)
        pltpu.async_copy(qr_hbm.at[pl.ds(ec, S)], qr_v, lsem)
        pltpu.async_copy(qv_hbm.at[pl.ds(ec, S)], qv_v, lsem)

    def wait_qn_fire_gathers(p):
        qn_v, _, _, rows_v, nsem, _, gsem = bufs[p]
        pltpu.make_async_copy(qn_hbm.at[pl.ds(0, S)], qn_v, nsem).wait()
        for j in range(NGPS):
            pltpu.async_copy(ut_hbm.at[qn_v.at[pl.ds(j * W, W)]],
                             rows_v.at[pl.ds(j * W, W), :], gsem)

    def wait_rest(p):
        qn_v, qr_v, qv_v, rows_v, _, lsem, gsem = bufs[p]
        pltpu.make_async_copy(qr_hbm.at[pl.ds(0, S)], qr_v, lsem).wait()
        pltpu.make_async_copy(qv_hbm.at[pl.ds(0, S)], qv_v, lsem).wait()
        for j in range(NGPS):
            pltpu.make_async_copy(ut_hbm.at[qn_v.at[pl.ds(j * W, W)]],
                                  rows_v.at[pl.ds(j * W, W), :], gsem).wait()

    @pl.loop(0, KMAX)
    def _round(k):
        blk = wid + NTILES * k

        @pl.when(blk < NB)
        def _process():
            r0 = blk * RPB
            e0 = jnp.max(plsc.load_gather(bnd_v, [jnp.full((16,), blk, jnp.int32)]))
            e1 = jnp.max(plsc.load_gather(bnd_v, [jnp.full((16,), blk + 1, jnp.int32)]))

            zeros16 = jnp.zeros((16,), jnp.float32)

            @pl.loop(0, B)
            def _zb(bi):
                @pl.loop(0, RPB // 16)
                def _zj(j):
                    acc_v[bi, pl.ds(j * 16, 16)] = zeros16

            e_base = pl.multiple_of(e0 - lax.rem(e0, 16), 16)
            nsc = lax.div(e1 - e_base + (S - 1), S)
            e0f = jnp.full((16,), e0, jnp.int32)
            e1f = jnp.full((16,), e1, jnp.int32)
            r0f = jnp.full((16,), r0, jnp.int32)

            def ecof(s):
                return pl.multiple_of(e_base + s * S, 16)

            def compute(p, s):
                _, qr_v, qv_v, rows_v, _, _, _ = bufs[p]
                ec = ecof(s)

                @pl.loop(0, NWINS)
                def _window(wd):
                    base = wd * 16
                    wvec = qv_v[pl.ds(base, 16)]
                    rvec = qr_v[pl.ds(base, 16)]
                    rl = rvec - r0f
                    gid = jnp.full((16,), ec + base, jnp.int32) + iota
                    mask = (gid >= e0f) & (gid < e1f)
                    eoff = jnp.full((16,), base, jnp.int32) + iota
                    for t in range(B):
                        btl = (iota + jnp.int32(t)) & jnp.int32(B - 1)
                        col = plsc.load_gather(rows_v, [eoff, btl])
                        plsc.addupdate_scatter(acc_v, [btl, rl], col * wvec,
                                               mask=mask)

            fire_linear(0, ecof(0))
            wait_qn_fire_gathers(0)

            @pl.when(1 < nsc)
            def _pb():
                fire_linear(1, ecof(1))

            def _pair(i, carry):
                s0 = 2 * i
                s1 = s0 + 1
                wait_rest(0)

                @pl.when(s1 < nsc)
                def _gb():
                    wait_qn_fire_gathers(1)

                compute(0, s0)

                @pl.when(s1 + 1 < nsc)
                def _la():
                    fire_linear(0, ecof(s1 + 1))

                @pl.when(s1 < nsc)
                def _bphase():
                    wait_rest(1)

                    @pl.when(s1 + 1 < nsc)
                    def _ga():
                        wait_qn_fire_gathers(0)

                    compute(1, s1)

                    @pl.when(s1 + 2 < nsc)
                    def _lb():
                        fire_linear(1, ecof(s1 + 2))

                return carry

            lax.fori_loop(0, lax.div(nsc + 1, 2), _pair, 0)

            copies = [
                pltpu.async_copy(acc_v.at[b, pl.ds(0, RPB)],
                                 out_hbm.at[b, pl.ds(r0, RPB)],
                                 dma_sem)
                for b in range(B)
            ]
            for c in copies:
                c.wait()


@jax.jit
def kernel(U, Qrow, Qneigh, Qval):
    UT = jnp.transpose(U)

    row_starts = jnp.arange(NB + 1, dtype=jnp.int32) * RPB
    bounds = jnp.searchsorted(Qrow, row_starts, side="left").astype(jnp.int32)
    bounds = jnp.pad(bounds, (0, BND_PAD - (NB + 1)), constant_values=NNZ)

    qn = jnp.pad(Qneigh, (0, PADE), constant_values=0)
    qr = jnp.pad(Qrow, (0, PADE), constant_values=NUM_ROWS)
    qv = jnp.pad(Qval, (0, PADE), constant_values=0.0)

    mesh = plsc.VectorSubcoreMesh(core_axis_name="c", subcore_axis_name="s")
    cp = pltpu.CompilerParams()
    if "needs_layout_passes" in pltpu.CompilerParams.__dataclass_fields__:
        cp = dataclasses.replace(cp, needs_layout_passes=False)
    cp = dataclasses.replace(cp, use_tc_tiling_on_sc=False)
    out = pl.kernel(
        _sc_body,
        compiler_params=cp,
        out_type=jax.ShapeDtypeStruct((B, NUM_ROWS), jnp.float32),
        mesh=mesh,
        scratch_types=[
            pltpu.VMEM((BND_PAD,), jnp.int32),
            pltpu.VMEM((S,), jnp.int32),
            pltpu.VMEM((S,), jnp.int32),
            pltpu.VMEM((S,), jnp.float32),
            pltpu.VMEM((S, B), jnp.float32),
            pltpu.VMEM((S,), jnp.int32),
            pltpu.VMEM((S,), jnp.int32),
            pltpu.VMEM((S,), jnp.float32),
            pltpu.VMEM((S, B), jnp.float32),
            pltpu.VMEM((B, ACC_PITCH), jnp.float32),
            pltpu.SemaphoreType.DMA,
            pltpu.SemaphoreType.DMA,
            pltpu.SemaphoreType.DMA,
            pltpu.SemaphoreType.DMA,
            pltpu.SemaphoreType.DMA,
            pltpu.SemaphoreType.DMA,
            pltpu.SemaphoreType.DMA,
        ],
    )(UT, qn, qr, qv, bounds)

    return jnp.reshape(out, (B, N_OUT, FP_LEN))


# speedup vs baseline: 8.1676x; 1.0207x over previous
import dataclasses

import jax
import jax.numpy as jnp
from jax import lax
from jax.experimental import pallas as pl
from jax.experimental.pallas import tpu as pltpu
from jax.experimental.pallas import tpu_sc as plsc

B = 32
N_POINTS = 100000
N_OUT = 50000
FP_LEN = 4
NNZ = 1600000
NUM_ROWS = N_OUT * FP_LEN

RPB = 400
ACC_PITCH = 401
NB = NUM_ROWS // RPB
NTILES = 32
KMAX = -(-NB // NTILES)
W = 128
S = 1024
NGPS = S // W
NWINS = S // 16
BND_PAD = 512
ECMAX = NNZ - S


def _sc_body(ut_hbm, qn_hbm, qr_hbm, qv_hbm, bnd_hbm, out_hbm,
             bnd_v, qn_a, qr_a, qv_a, rows_a, qn_b, qr_b, qv_b, rows_b,
             acc_v, nsem_a, lsem_a, gsem_a, nsem_b, lsem_b, gsem_b, dma_sem):
    iota = lax.iota(jnp.int32, 16)
    wid = lax.axis_index("s") * 2 + lax.axis_index("c")

    pltpu.async_copy(bnd_hbm, bnd_v, dma_sem).wait()

    bufs = {
        0: (qn_a, qr_a, qv_a, rows_a, nsem_a, lsem_a, gsem_a),
        1: (qn_b, qr_b, qv_b, rows_b, nsem_b, lsem_b, gsem_b),
    }

    def fire_linear(p, ec):
        ecc = pl.multiple_of(jnp.minimum(ec, jnp.int32(ECMAX)), 16)
        qn_v, qr_v, qv_v, _, nsem, lsem, _ = bufs[p]
        pltpu.async_copy(qn_hbm.at[pl.ds(ecc, S)], qn_v, nsem)
        pltpu.async_copy(qr_hbm.at[pl.ds(ecc, S)], qr_v, lsem)
        pltpu.async_copy(qv_hbm.at[pl.ds(ecc, S)], qv_v, lsem)

    def wait_qn_fire_gathers(p):
        qn_v, _, _, rows_v, nsem, _, gsem = bufs[p]
        pltpu.make_async_copy(qn_hbm.at[pl.ds(0, S)], qn_v, nsem).wait()
        for j in range(NGPS):
            pltpu.async_copy(ut_hbm.at[qn_v.at[pl.ds(j * W, W)]],
                             rows_v.at[pl.ds(j * W, W), :], gsem)

    def wait_rest(p):
        qn_v, qr_v, qv_v, rows_v, _, lsem, gsem = bufs[p]
        pltpu.make_async_copy(qr_hbm.at[pl.ds(0, S)], qr_v, lsem).wait()
        pltpu.make_async_copy(qv_hbm.at[pl.ds(0, S)], qv_v, lsem).wait()
        for j in range(NGPS):
            pltpu.make_async_copy(ut_hbm.at[qn_v.at[pl.ds(j * W, W)]],
                                  rows_v.at[pl.ds(j * W, W), :], gsem).wait()

    @pl.loop(0, KMAX)
    def _round(k):
        blk = wid + NTILES * k

        @pl.when(blk < NB)
        def _process():
            r0 = blk * RPB
            e0 = jnp.max(plsc.load_gather(bnd_v, [jnp.full((16,), blk, jnp.int32)]))
            e1 = jnp.max(plsc.load_gather(bnd_v, [jnp.full((16,), blk + 1, jnp.int32)]))

            zeros16 = jnp.zeros((16,), jnp.float32)

            @pl.loop(0, B)
            def _zb(bi):
                @pl.loop(0, RPB // 16)
                def _zj(j):
                    acc_v[bi, pl.ds(j * 16, 16)] = zeros16

            e_base = pl.multiple_of(e0 - lax.rem(e0, 16), 16)
            nsc = lax.div(e1 - e_base + (S - 1), S)
            e0f = jnp.full((16,), e0, jnp.int32)
            e1f = jnp.full((16,), e1, jnp.int32)
            r0f = jnp.full((16,), r0, jnp.int32)

            def ecof(s):
                return pl.multiple_of(e_base + s * S, 16)

            def compute(p, s):
                _, qr_v, qv_v, rows_v, _, _, _ = bufs[p]
                ec = ecof(s)
                ecc = pl.multiple_of(jnp.minimum(ec, jnp.int32(ECMAX)), 16)
                d = ec - ecc

                @pl.loop(0, NWINS)
                def _window(wd):
                    base = wd * 16
                    gid = jnp.full((16,), ec + base, jnp.int32) + iota
                    mask = (gid >= e0f) & (gid < e1f)
                    eoff = jnp.minimum(
                        jnp.full((16,), d + base, jnp.int32) + iota,
                        jnp.int32(S - 1))
                    wvec = plsc.load_gather(qv_v, [eoff])
                    rvec = plsc.load_gather(qr_v, [eoff])
                    rl = rvec - r0f
                    for t in range(B):
                        btl = (iota + jnp.int32(t)) & jnp.int32(B - 1)
                        col = plsc.load_gather(rows_v, [eoff, btl])
                        plsc.addupdate_scatter(acc_v, [btl, rl], col * wvec,
                                               mask=mask)

            fire_linear(0, ecof(0))
            wait_qn_fire_gathers(0)

            @pl.when(1 < nsc)
            def _pb():
                fire_linear(1, ecof(1))

            def _pair(i, carry):
                s0 = 2 * i
                s1 = s0 + 1
                wait_rest(0)

                @pl.when(s1 < nsc)
                def _gb():
                    wait_qn_fire_gathers(1)

                compute(0, s0)

                @pl.when(s1 + 1 < nsc)
                def _la():
                    fire_linear(0, ecof(s1 + 1))

                @pl.when(s1 < nsc)
                def _bphase():
                    wait_rest(1)

                    @pl.when(s1 + 1 < nsc)
                    def _ga():
                        wait_qn_fire_gathers(0)

                    compute(1, s1)

                    @pl.when(s1 + 2 < nsc)
                    def _lb():
                        fire_linear(1, ecof(s1 + 2))

                return carry

            lax.fori_loop(0, lax.div(nsc + 1, 2), _pair, 0)

            copies = [
                pltpu.async_copy(acc_v.at[b, pl.ds(0, RPB)],
                                 out_hbm.at[b, pl.ds(r0, RPB)],
                                 dma_sem)
                for b in range(B)
            ]
            for c in copies:
                c.wait()


@jax.jit
def kernel(U, Qrow, Qneigh, Qval):
    UT = jnp.transpose(U)

    row_starts = jnp.arange(NB + 1, dtype=jnp.int32) * RPB
    bounds = jnp.searchsorted(Qrow, row_starts, side="left").astype(jnp.int32)
    bounds = jnp.pad(bounds, (0, BND_PAD - (NB + 1)), constant_values=NNZ)

    mesh = plsc.VectorSubcoreMesh(core_axis_name="c", subcore_axis_name="s")
    cp = pltpu.CompilerParams()
    if "needs_layout_passes" in pltpu.CompilerParams.__dataclass_fields__:
        cp = dataclasses.replace(cp, needs_layout_passes=False)
    cp = dataclasses.replace(cp, use_tc_tiling_on_sc=False)
    out = pl.kernel(
        _sc_body,
        compiler_params=cp,
        out_type=jax.ShapeDtypeStruct((B, NUM_ROWS), jnp.float32),
        mesh=mesh,
        scratch_types=[
            pltpu.VMEM((BND_PAD,), jnp.int32),
            pltpu.VMEM((S,), jnp.int32),
            pltpu.VMEM((S,), jnp.int32),
            pltpu.VMEM((S,), jnp.float32),
            pltpu.VMEM((S, B), jnp.float32),
            pltpu.VMEM((S,), jnp.int32),
            pltpu.VMEM((S,), jnp.int32),
            pltpu.VMEM((S,), jnp.float32),
            pltpu.VMEM((S, B), jnp.float32),
            pltpu.VMEM((B, ACC_PITCH), jnp.float32),
            pltpu.SemaphoreType.DMA,
            pltpu.SemaphoreType.DMA,
            pltpu.SemaphoreType.DMA,
            pltpu.SemaphoreType.DMA,
            pltpu.SemaphoreType.DMA,
            pltpu.SemaphoreType.DMA,
            pltpu.SemaphoreType.DMA,
        ],
    )(UT, Qneigh, Qrow, Qval, bounds)

    return jnp.reshape(out, (B, N_OUT, FP_LEN))


# speedup vs baseline: 8.1728x; 1.0006x over previous
import dataclasses

import jax
import jax.numpy as jnp
from jax import lax
from jax.experimental import pallas as pl
from jax.experimental.pallas import tpu as pltpu
from jax.experimental.pallas import tpu_sc as plsc

B = 32
N_POINTS = 100000
N_OUT = 50000
FP_LEN = 4
NNZ = 1600000
NUM_ROWS = N_OUT * FP_LEN

RPB = 400
ACC_PITCH = 401
NB = NUM_ROWS // RPB
NTILES = 32
KMAX = -(-NB // NTILES)
W = 128
S = 1024
NGPS = S // W
NWINS = S // 16
BND_PAD = 512
ECMAX = NNZ - S


def _sc_body(ut_hbm, qn_hbm, qr_hbm, qv_hbm, bnd_hbm, out_hbm,
             bnd_v, qn_a, qr_a, qv_a, rows_a, qn_b, qr_b, qv_b, rows_b,
             acc_v, nsem_a, lsem_a, gsem_a, nsem_b, lsem_b, gsem_b, dma_sem):
    iota = lax.iota(jnp.int32, 16)
    wid = lax.axis_index("s") * 2 + lax.axis_index("c")

    pltpu.async_copy(bnd_hbm, bnd_v, dma_sem).wait()

    bufs = {
        0: (qn_a, qr_a, qv_a, rows_a, nsem_a, lsem_a, gsem_a),
        1: (qn_b, qr_b, qv_b, rows_b, nsem_b, lsem_b, gsem_b),
    }

    def fire_linear(p, ec):
        ecc = pl.multiple_of(jnp.minimum(ec, jnp.int32(ECMAX)), 16)
        qn_v, qr_v, qv_v, _, nsem, lsem, _ = bufs[p]
        pltpu.async_copy(qn_hbm.at[pl.ds(ecc, S)], qn_v, nsem)
        pltpu.async_copy(qr_hbm.at[pl.ds(ecc, S)], qr_v, lsem)
        pltpu.async_copy(qv_hbm.at[pl.ds(ecc, S)], qv_v, lsem)

    def wait_qn_fire_gathers(p):
        qn_v, _, _, rows_v, nsem, _, gsem = bufs[p]
        pltpu.make_async_copy(qn_hbm.at[pl.ds(0, S)], qn_v, nsem).wait()
        for j in range(NGPS):
            pltpu.async_copy(ut_hbm.at[qn_v.at[pl.ds(j * W, W)]],
                             rows_v.at[pl.ds(j * W, W), :], gsem)

    def wait_rest(p):
        qn_v, qr_v, qv_v, rows_v, _, lsem, gsem = bufs[p]
        pltpu.make_async_copy(qr_hbm.at[pl.ds(0, S)], qr_v, lsem).wait()
        pltpu.make_async_copy(qv_hbm.at[pl.ds(0, S)], qv_v, lsem).wait()
        for j in range(NGPS):
            pltpu.make_async_copy(ut_hbm.at[qn_v.at[pl.ds(j * W, W)]],
                                  rows_v.at[pl.ds(j * W, W), :], gsem).wait()

    @pl.loop(0, KMAX)
    def _round(k):
        blk = wid + NTILES * k

        @pl.when(blk < NB)
        def _process():
            r0 = blk * RPB
            e0 = jnp.max(plsc.load_gather(bnd_v, [jnp.full((16,), blk, jnp.int32)]))
            e1 = jnp.max(plsc.load_gather(bnd_v, [jnp.full((16,), blk + 1, jnp.int32)]))

            zeros16 = jnp.zeros((16,), jnp.float32)

            @pl.loop(0, B)
            def _zb(bi):
                @pl.loop(0, RPB // 16)
                def _zj(j):
                    acc_v[bi, pl.ds(j * 16, 16)] = zeros16

            e_base = pl.multiple_of(e0 - lax.rem(e0, 16), 16)
            nsc = lax.div(e1 - e_base + (S - 1), S)
            e0f = jnp.full((16,), e0, jnp.int32)
            e1f = jnp.full((16,), e1, jnp.int32)
            r0f = jnp.full((16,), r0, jnp.int32)

            def ecof(s):
                return pl.multiple_of(e_base + s * S, 16)

            def compute(p, s):
                _, qr_v, qv_v, rows_v, _, _, _ = bufs[p]
                ec = ecof(s)
                ecc = pl.multiple_of(jnp.minimum(ec, jnp.int32(ECMAX)), 16)
                d = ec - ecc

                @pl.loop(0, NWINS)
                def _window(wd):
                    base = wd * 16
                    gid = jnp.full((16,), ec + base, jnp.int32) + iota
                    mask = (gid >= e0f) & (gid < e1f)
                    eoff = jnp.minimum(
                        jnp.full((16,), d + base, jnp.int32) + iota,
                        jnp.int32(S - 1))
                    wvec = plsc.load_gather(qv_v, [eoff])
                    rvec = plsc.load_gather(qr_v, [eoff])
                    rl = rvec - r0f
                    for t in range(B):
                        btl = (iota + jnp.int32(t)) & jnp.int32(B - 1)
                        col = plsc.load_gather(rows_v, [eoff, btl])
                        plsc.addupdate_scatter(acc_v, [btl, rl], col * wvec,
                                               mask=mask)

            fire_linear(0, ecof(0))
            wait_qn_fire_gathers(0)

            @pl.when(1 < nsc)
            def _pb():
                fire_linear(1, ecof(1))

            def _pair(i, carry):
                s0 = 2 * i
                s1 = s0 + 1
                wait_rest(0)

                @pl.when(s1 < nsc)
                def _gb():
                    wait_qn_fire_gathers(1)

                compute(0, s0)

                @pl.when(s1 + 1 < nsc)
                def _la():
                    fire_linear(0, ecof(s1 + 1))

                @pl.when(s1 < nsc)
                def _bphase():
                    wait_rest(1)

                    @pl.when(s1 + 1 < nsc)
                    def _ga():
                        wait_qn_fire_gathers(0)

                    compute(1, s1)

                    @pl.when(s1 + 2 < nsc)
                    def _lb():
                        fire_linear(1, ecof(s1 + 2))

                return carry

            lax.fori_loop(0, lax.div(nsc + 1, 2), _pair, 0)

            copies = [
                pltpu.async_copy(acc_v.at[b, pl.ds(0, RPB)],
                                 out_hbm.at[b, pl.ds(r0, RPB)],
                                 dma_sem)
                for b in range(B)
            ]
            for c in copies:
                c.wait()


@jax.jit
def kernel(U, Qrow, Qneigh, Qval):
    UT = jnp.transpose(U)

    row_starts = jnp.arange(NB + 1, dtype=jnp.int32) * RPB
    bounds = jnp.searchsorted(Qrow, row_starts, side="left",
                              method="scan_unrolled").astype(jnp.int32)
    bounds = jnp.pad(bounds, (0, BND_PAD - (NB + 1)), constant_values=NNZ)

    mesh = plsc.VectorSubcoreMesh(core_axis_name="c", subcore_axis_name="s")
    cp = pltpu.CompilerParams()
    if "needs_layout_passes" in pltpu.CompilerParams.__dataclass_fields__:
        cp = dataclasses.replace(cp, needs_layout_passes=False)
    cp = dataclasses.replace(cp, use_tc_tiling_on_sc=False)
    out = pl.kernel(
        _sc_body,
        compiler_params=cp,
        out_type=jax.ShapeDtypeStruct((B, NUM_ROWS), jnp.float32),
        mesh=mesh,
        scratch_types=[
            pltpu.VMEM((BND_PAD,), jnp.int32),
            pltpu.VMEM((S,), jnp.int32),
            pltpu.VMEM((S,), jnp.int32),
            pltpu.VMEM((S,), jnp.float32),
            pltpu.VMEM((S, B), jnp.float32),
            pltpu.VMEM((S,), jnp.int32),
            pltpu.VMEM((S,), jnp.int32),
            pltpu.VMEM((S,), jnp.float32),
            pltpu.VMEM((S, B), jnp.float32),
            pltpu.VMEM((B, ACC_PITCH), jnp.float32),
            pltpu.SemaphoreType.DMA,
            pltpu.SemaphoreType.DMA,
            pltpu.SemaphoreType.DMA,
            pltpu.SemaphoreType.DMA,
            pltpu.SemaphoreType.DMA,
            pltpu.SemaphoreType.DMA,
            pltpu.SemaphoreType.DMA,
        ],
    )(UT, Qneigh, Qrow, Qval, bounds)

    return jnp.reshape(out, (B, N_OUT, FP_LEN))


# speedup vs baseline: 9.3480x; 1.1438x over previous
import dataclasses

import jax
import jax.numpy as jnp
from jax import lax
from jax.experimental import pallas as pl
from jax.experimental.pallas import tpu as pltpu
from jax.experimental.pallas import tpu_sc as plsc

B = 32
N_POINTS = 100000
N_OUT = 50000
FP_LEN = 4
NNZ = 1600000
NUM_ROWS = N_OUT * FP_LEN

RPB = 800
ACC_PITCH = 801
NB = NUM_ROWS // RPB
NTILES = 32
KMAX = -(-NB // NTILES)
W = 128
S = 1024
NGPS = S // W
NWINS = S // 16
BND_PAD = 512
ECMAX = NNZ - S


def _sc_body(ut_hbm, qn_hbm, qr_hbm, qv_hbm, bnd_hbm, out_hbm,
             bnd_v, qn_a, qr_a, qv_a, rows_a, qn_b, qr_b, qv_b, rows_b,
             acc_v, nsem_a, lsem_a, gsem_a, nsem_b, lsem_b, gsem_b, dma_sem):
    iota = lax.iota(jnp.int32, 16)
    wid = lax.axis_index("s") * 2 + lax.axis_index("c")

    pltpu.async_copy(bnd_hbm, bnd_v, dma_sem).wait()

    bufs = {
        0: (qn_a, qr_a, qv_a, rows_a, nsem_a, lsem_a, gsem_a),
        1: (qn_b, qr_b, qv_b, rows_b, nsem_b, lsem_b, gsem_b),
    }

    def fire_linear(p, ec):
        ecc = pl.multiple_of(jnp.minimum(ec, jnp.int32(ECMAX)), 16)
        qn_v, qr_v, qv_v, _, nsem, lsem, _ = bufs[p]
        pltpu.async_copy(qn_hbm.at[pl.ds(ecc, S)], qn_v, nsem)
        pltpu.async_copy(qr_hbm.at[pl.ds(ecc, S)], qr_v, lsem)
        pltpu.async_copy(qv_hbm.at[pl.ds(ecc, S)], qv_v, lsem)

    def wait_qn_fire_gathers(p):
        qn_v, _, _, rows_v, nsem, _, gsem = bufs[p]
        pltpu.make_async_copy(qn_hbm.at[pl.ds(0, S)], qn_v, nsem).wait()
        for j in range(NGPS):
            pltpu.async_copy(ut_hbm.at[qn_v.at[pl.ds(j * W, W)]],
                             rows_v.at[pl.ds(j * W, W), :], gsem)

    def wait_rest(p):
        qn_v, qr_v, qv_v, rows_v, _, lsem, gsem = bufs[p]
        pltpu.make_async_copy(qr_hbm.at[pl.ds(0, S)], qr_v, lsem).wait()
        pltpu.make_async_copy(qv_hbm.at[pl.ds(0, S)], qv_v, lsem).wait()
        for j in range(NGPS):
            pltpu.make_async_copy(ut_hbm.at[qn_v.at[pl.ds(j * W, W)]],
                                  rows_v.at[pl.ds(j * W, W), :], gsem).wait()

    @pl.loop(0, KMAX)
    def _round(k):
        blk = wid + NTILES * k

        @pl.when(blk < NB)
        def _process():
            r0 = blk * RPB
            e0 = jnp.max(plsc.load_gather(bnd_v, [jnp.full((16,), blk, jnp.int32)]))
            e1 = jnp.max(plsc.load_gather(bnd_v, [jnp.full((16,), blk + 1, jnp.int32)]))

            zeros16 = jnp.zeros((16,), jnp.float32)

            @pl.loop(0, B)
            def _zb(bi):
                @pl.loop(0, RPB // 16)
                def _zj(j):
                    acc_v[bi, pl.ds(j * 16, 16)] = zeros16

            e_base = pl.multiple_of(e0 - lax.rem(e0, 16), 16)
            nsc = lax.div(e1 - e_base + (S - 1), S)
            e0f = jnp.full((16,), e0, jnp.int32)
            e1f = jnp.full((16,), e1, jnp.int32)
            r0f = jnp.full((16,), r0, jnp.int32)

            def ecof(s):
                return pl.multiple_of(e_base + s * S, 16)

            def compute(p, s):
                _, qr_v, qv_v, rows_v, _, _, _ = bufs[p]
                ec = ecof(s)
                ecc = pl.multiple_of(jnp.minimum(ec, jnp.int32(ECMAX)), 16)
                d = ec - ecc

                @pl.loop(0, NWINS)
                def _window(wd):
                    base = wd * 16
                    gid = jnp.full((16,), ec + base, jnp.int32) + iota
                    mask = (gid >= e0f) & (gid < e1f)
                    eoff = jnp.minimum(
                        jnp.full((16,), d + base, jnp.int32) + iota,
                        jnp.int32(S - 1))
                    wvec = plsc.load_gather(qv_v, [eoff])
                    rvec = plsc.load_gather(qr_v, [eoff])
                    rl = rvec - r0f
                    for t in range(B):
                        btl = (iota + jnp.int32(t)) & jnp.int32(B - 1)
                        col = plsc.load_gather(rows_v, [eoff, btl])
                        plsc.addupdate_scatter(acc_v, [btl, rl], col * wvec,
                                               mask=mask)

            fire_linear(0, ecof(0))
            wait_qn_fire_gathers(0)

            @pl.when(1 < nsc)
            def _pb():
                fire_linear(1, ecof(1))

            def _pair(i, carry):
                s0 = 2 * i
                s1 = s0 + 1
                wait_rest(0)

                @pl.when(s1 < nsc)
                def _gb():
                    wait_qn_fire_gathers(1)

                compute(0, s0)

                @pl.when(s1 + 1 < nsc)
                def _la():
                    fire_linear(0, ecof(s1 + 1))

                @pl.when(s1 < nsc)
                def _bphase():
                    wait_rest(1)

                    @pl.when(s1 + 1 < nsc)
                    def _ga():
                        wait_qn_fire_gathers(0)

                    compute(1, s1)

                    @pl.when(s1 + 2 < nsc)
                    def _lb():
                        fire_linear(1, ecof(s1 + 2))

                return carry

            lax.fori_loop(0, lax.div(nsc + 1, 2), _pair, 0)

            copies = [
                pltpu.async_copy(acc_v.at[b, pl.ds(0, RPB)],
                                 out_hbm.at[b, pl.ds(r0, RPB)],
                                 dma_sem)
                for b in range(B)
            ]
            for c in copies:
                c.wait()


@jax.jit
def kernel(U, Qrow, Qneigh, Qval):
    UT = jnp.transpose(U)

    row_starts = jnp.arange(NB + 1, dtype=jnp.int32) * RPB
    bounds = jnp.searchsorted(Qrow, row_starts, side="left",
                              method="scan_unrolled").astype(jnp.int32)
    bounds = jnp.pad(bounds, (0, BND_PAD - (NB + 1)), constant_values=NNZ)

    mesh = plsc.VectorSubcoreMesh(core_axis_name="c", subcore_axis_name="s")
    cp = pltpu.CompilerParams()
    if "needs_layout_passes" in pltpu.CompilerParams.__dataclass_fields__:
        cp = dataclasses.replace(cp, needs_layout_passes=False)
    cp = dataclasses.replace(cp, use_tc_tiling_on_sc=False)
    out = pl.kernel(
        _sc_body,
        compiler_params=cp,
        out_type=jax.ShapeDtypeStruct((B, NUM_ROWS), jnp.float32),
        mesh=mesh,
        scratch_types=[
            pltpu.VMEM((BND_PAD,), jnp.int32),
            pltpu.VMEM((S,), jnp.int32),
            pltpu.VMEM((S,), jnp.int32),
            pltpu.VMEM((S,), jnp.float32),
            pltpu.VMEM((S, B), jnp.float32),
            pltpu.VMEM((S,), jnp.int32),
            pltpu.VMEM((S,), jnp.int32),
            pltpu.VMEM((S,), jnp.float32),
            pltpu.VMEM((S, B), jnp.float32),
            pltpu.VMEM((B, ACC_PITCH), jnp.float32),
            pltpu.SemaphoreType.DMA,
            pltpu.SemaphoreType.DMA,
            pltpu.SemaphoreType.DMA,
            pltpu.SemaphoreType.DMA,
            pltpu.SemaphoreType.DMA,
            pltpu.SemaphoreType.DMA,
            pltpu.SemaphoreType.DMA,
        ],
    )(UT, Qneigh, Qrow, Qval, bounds)

    return jnp.reshape(out, (B, N_OUT, FP_LEN))


# speedup vs baseline: 10.4377x; 1.1166x over previous
import dataclasses

import jax
import jax.numpy as jnp
from jax import lax
from jax.experimental import pallas as pl
from jax.experimental.pallas import tpu as pltpu
from jax.experimental.pallas import tpu_sc as plsc

B = 32
N_POINTS = 100000
N_OUT = 50000
FP_LEN = 4
NNZ = 1600000
NUM_ROWS = N_OUT * FP_LEN

RPB = 1600
ACC_PITCH = 1601
NB = NUM_ROWS // RPB
NTILES = 32
KMAX = -(-NB // NTILES)
W = 128
S = 1024
NGPS = S // W
NWINS = S // 16
BND_PAD = 512
ECMAX = NNZ - S


def _sc_body(ut_hbm, qn_hbm, qr_hbm, qv_hbm, bnd_hbm, out_hbm,
             bnd_v, qn_a, qr_a, qv_a, rows_a, qn_b, qr_b, qv_b, rows_b,
             acc_v, nsem_a, lsem_a, gsem_a, nsem_b, lsem_b, gsem_b, dma_sem):
    iota = lax.iota(jnp.int32, 16)
    wid = lax.axis_index("s") * 2 + lax.axis_index("c")

    pltpu.async_copy(bnd_hbm, bnd_v, dma_sem).wait()

    bufs = {
        0: (qn_a, qr_a, qv_a, rows_a, nsem_a, lsem_a, gsem_a),
        1: (qn_b, qr_b, qv_b, rows_b, nsem_b, lsem_b, gsem_b),
    }

    def fire_linear(p, ec):
        ecc = pl.multiple_of(jnp.minimum(ec, jnp.int32(ECMAX)), 16)
        qn_v, qr_v, qv_v, _, nsem, lsem, _ = bufs[p]
        pltpu.async_copy(qn_hbm.at[pl.ds(ecc, S)], qn_v, nsem)
        pltpu.async_copy(qr_hbm.at[pl.ds(ecc, S)], qr_v, lsem)
        pltpu.async_copy(qv_hbm.at[pl.ds(ecc, S)], qv_v, lsem)

    def wait_qn_fire_gathers(p):
        qn_v, _, _, rows_v, nsem, _, gsem = bufs[p]
        pltpu.make_async_copy(qn_hbm.at[pl.ds(0, S)], qn_v, nsem).wait()
        for j in range(NGPS):
            pltpu.async_copy(ut_hbm.at[qn_v.at[pl.ds(j * W, W)]],
                             rows_v.at[pl.ds(j * W, W), :], gsem)

    def wait_rest(p):
        qn_v, qr_v, qv_v, rows_v, _, lsem, gsem = bufs[p]
        pltpu.make_async_copy(qr_hbm.at[pl.ds(0, S)], qr_v, lsem).wait()
        pltpu.make_async_copy(qv_hbm.at[pl.ds(0, S)], qv_v, lsem).wait()
        for j in range(NGPS):
            pltpu.make_async_copy(ut_hbm.at[qn_v.at[pl.ds(j * W, W)]],
                                  rows_v.at[pl.ds(j * W, W), :], gsem).wait()

    @pl.loop(0, KMAX)
    def _round(k):
        blk = wid + NTILES * k

        @pl.when(blk < NB)
        def _process():
            r0 = blk * RPB
            e0 = jnp.max(plsc.load_gather(bnd_v, [jnp.full((16,), blk, jnp.int32)]))
            e1 = jnp.max(plsc.load_gather(bnd_v, [jnp.full((16,), blk + 1, jnp.int32)]))

            zeros16 = jnp.zeros((16,), jnp.float32)

            @pl.loop(0, B)
            def _zb(bi):
                @pl.loop(0, RPB // 16)
                def _zj(j):
                    acc_v[bi, pl.ds(j * 16, 16)] = zeros16

            e_base = pl.multiple_of(e0 - lax.rem(e0, 16), 16)
            nsc = lax.div(e1 - e_base + (S - 1), S)
            e0f = jnp.full((16,), e0, jnp.int32)
            e1f = jnp.full((16,), e1, jnp.int32)
            r0f = jnp.full((16,), r0, jnp.int32)

            def ecof(s):
                return pl.multiple_of(e_base + s * S, 16)

            def compute(p, s):
                _, qr_v, qv_v, rows_v, _, _, _ = bufs[p]
                ec = ecof(s)
                ecc = pl.multiple_of(jnp.minimum(ec, jnp.int32(ECMAX)), 16)
                d = ec - ecc

                @pl.loop(0, NWINS)
                def _window(wd):
                    base = wd * 16
                    gid = jnp.full((16,), ec + base, jnp.int32) + iota
                    mask = (gid >= e0f) & (gid < e1f)
                    eoff = jnp.minimum(
                        jnp.full((16,), d + base, jnp.int32) + iota,
                        jnp.int32(S - 1))
                    wvec = plsc.load_gather(qv_v, [eoff])
                    rvec = plsc.load_gather(qr_v, [eoff])
                    rl = rvec - r0f
                    for t in range(B):
                        btl = (iota + jnp.int32(t)) & jnp.int32(B - 1)
                        col = plsc.load_gather(rows_v, [eoff, btl])
                        plsc.addupdate_scatter(acc_v, [btl, rl], col * wvec,
                                               mask=mask)

            fire_linear(0, ecof(0))
            wait_qn_fire_gathers(0)

            @pl.when(1 < nsc)
            def _pb():
                fire_linear(1, ecof(1))

            def _pair(i, carry):
                s0 = 2 * i
                s1 = s0 + 1
                wait_rest(0)

                @pl.when(s1 < nsc)
                def _gb():
                    wait_qn_fire_gathers(1)

                compute(0, s0)

                @pl.when(s1 + 1 < nsc)
                def _la():
                    fire_linear(0, ecof(s1 + 1))

                @pl.when(s1 < nsc)
                def _bphase():
                    wait_rest(1)

                    @pl.when(s1 + 1 < nsc)
                    def _ga():
                        wait_qn_fire_gathers(0)

                    compute(1, s1)

                    @pl.when(s1 + 2 < nsc)
                    def _lb():
                        fire_linear(1, ecof(s1 + 2))

                return carry

            lax.fori_loop(0, lax.div(nsc + 1, 2), _pair, 0)

            copies = [
                pltpu.async_copy(acc_v.at[b, pl.ds(0, RPB)],
                                 out_hbm.at[b, pl.ds(r0, RPB)],
                                 dma_sem)
                for b in range(B)
            ]
            for c in copies:
                c.wait()


@jax.jit
def kernel(U, Qrow, Qneigh, Qval):
    UT = jnp.transpose(U)

    row_starts = jnp.arange(NB + 1, dtype=jnp.int32) * RPB
    bounds = jnp.searchsorted(Qrow, row_starts, side="left",
                              method="scan_unrolled").astype(jnp.int32)
    bounds = jnp.pad(bounds, (0, BND_PAD - (NB + 1)), constant_values=NNZ)

    mesh = plsc.VectorSubcoreMesh(core_axis_name="c", subcore_axis_name="s")
    cp = pltpu.CompilerParams()
    if "needs_layout_passes" in pltpu.CompilerParams.__dataclass_fields__:
        cp = dataclasses.replace(cp, needs_layout_passes=False)
    cp = dataclasses.replace(cp, use_tc_tiling_on_sc=False)
    out = pl.kernel(
        _sc_body,
        compiler_params=cp,
        out_type=jax.ShapeDtypeStruct((B, NUM_ROWS), jnp.float32),
        mesh=mesh,
        scratch_types=[
            pltpu.VMEM((BND_PAD,), jnp.int32),
            pltpu.VMEM((S,), jnp.int32),
            pltpu.VMEM((S,), jnp.int32),
            pltpu.VMEM((S,), jnp.float32),
            pltpu.VMEM((S, B), jnp.float32),
            pltpu.VMEM((S,), jnp.int32),
            pltpu.VMEM((S,), jnp.int32),
            pltpu.VMEM((S,), jnp.float32),
            pltpu.VMEM((S, B), jnp.float32),
            pltpu.VMEM((B, ACC_PITCH), jnp.float32),
            pltpu.SemaphoreType.DMA,
            pltpu.SemaphoreType.DMA,
            pltpu.SemaphoreType.DMA,
            pltpu.SemaphoreType.DMA,
            pltpu.SemaphoreType.DMA,
            pltpu.SemaphoreType.DMA,
            pltpu.SemaphoreType.DMA,
        ],
    )(UT, Qneigh, Qrow, Qval, bounds)

    return jnp.reshape(out, (B, N_OUT, FP_LEN))
